# Initial kernel scaffold; baseline (speedup 1.0000x reference)
import jax
import jax.numpy as jnp
from jax.experimental import pallas as pl


def kernel(x, edge_index, edge_attr, W_init, b_init, W_edge, b_edge, eps, W1, b1, W2, b2, gamma, beta):
    raise NotImplementedError("write your pallas kernel here")


# speedup vs baseline: 2.0826x; 2.0826x over previous
import functools

import jax
import jax.numpy as jnp
from jax import lax
from jax.experimental import pallas as pl
from jax.experimental.pallas import tpu as pltpu
from jax.experimental.pallas import tpu_sc as plsc

NC = 2
NS = 16
LANES = 16


def _encode_body(x_ref, w_ref, b_ref, o_ref):
    o_ref[...] = (
        jnp.dot(x_ref[...], w_ref[...], preferred_element_type=jnp.float32)
        + b_ref[...]
    )


def _encode(x, w, b):
    n, d = x.shape
    return pl.pallas_call(
        _encode_body,
        out_shape=jax.ShapeDtypeStruct((n, d), jnp.float32),
    )(x, w, b)


def _edge_emb_body(a_ref, w_ref, b_ref, o_ref):
    o_ref[0] = (
        jnp.dot(a_ref[...], w_ref[0], preferred_element_type=jnp.float32)
        + b_ref[0]
    )


def _edge_emb(edge_attr, w_edge, b_edge, block_e):
    num_l, de, d = w_edge.shape
    e = edge_attr.shape[0]
    grid = (num_l, e // block_e)
    return pl.pallas_call(
        _edge_emb_body,
        grid=grid,
        in_specs=[
            pl.BlockSpec((block_e, de), lambda l, i: (i, 0)),
            pl.BlockSpec((1, de, d), lambda l, i: (l, 0, 0)),
            pl.BlockSpec((1, 1, d), lambda l, i: (l, 0, 0)),
        ],
        out_specs=pl.BlockSpec((1, block_e, d), lambda l, i: (l, i, 0)),
        out_shape=jax.ShapeDtypeStruct((num_l, e, d), jnp.float32),
    )(edge_attr, w_edge, b_edge.reshape(num_l, 1, d))


def _layer_body(h_ref, agg_ref, w1_ref, b1_ref, w2_ref, b2_ref, g_ref,
                bt_ref, eps_ref, o_ref):
    h = h_ref[...]
    agg = agg_ref[0] + agg_ref[1]
    z = h * (1.0 + eps_ref[0, 0]) + agg
    t = jnp.maximum(
        jnp.dot(z, w1_ref[...], preferred_element_type=jnp.float32)
        + b1_ref[...], 0.0)
    t = jnp.dot(t, w2_ref[...], preferred_element_type=jnp.float32) + b2_ref[...]
    mean = jnp.mean(t, axis=0, keepdims=True)
    var = jnp.mean((t - mean) * (t - mean), axis=0, keepdims=True)
    o_ref[...] = (t - mean) * lax.rsqrt(var + 1e-5) * g_ref[...] + bt_ref[...] + h


def _layer(h, agg, w1, b1, w2, b2, gamma, beta, eps_l):
    n, d = h.shape
    return pl.pallas_call(
        _layer_body,
        out_shape=jax.ShapeDtypeStruct((n, d), jnp.float32),
    )(h, agg, w1, b1, w2, b2, gamma, beta, eps_l)


def _sc_message_pass(h, emb, src, dst):
    n, d = h.shape
    e = src.shape[0]
    nw = NC * NS
    ew = e // nw
    chunk = 80
    nchunk = ew // chunk
    assert ew % chunk == 0 and e % nw == 0
    npt = (n // NS) // 8 * 8
    tail = n - npt * NS
    zrows = 208
    assert npt % zrows == 0 and tail % 8 == 0 and tail <= zrows
    groups = d // LANES

    mesh = plsc.VectorSubcoreMesh(core_axis_name="c", subcore_axis_name="s")

    @functools.partial(
        pl.kernel,
        out_type=jax.ShapeDtypeStruct((NC, n, d), jnp.float32),
        mesh=mesh,
        scratch_types=[
            pltpu.VMEM((chunk,), jnp.int32),
            pltpu.VMEM((chunk,), jnp.int32),
            pltpu.VMEM((chunk, d), jnp.float32),
            pltpu.VMEM((chunk, d), jnp.float32),
            pltpu.VMEM((zrows, d), jnp.float32),
            pltpu.VMEM_SHARED((n, d), jnp.float32),
            pltpu.SemaphoreType.DMA,
        ],
    )
    def body(h_hbm, emb_hbm, src_hbm, dst_hbm, out_hbm,
             srcv, dstv, rows, embv, zbuf, aggs, sem):
        cid = lax.axis_index("c")
        sid = lax.axis_index("s")
        wid = sid * NC + cid

        def zero_body(i, carry):
            for j in range(groups):
                zbuf[i, pl.ds(j * LANES, LANES)] = jnp.zeros(
                    (LANES,), jnp.float32)
            return carry
        lax.fori_loop(0, zrows, zero_body, 0)
        for k in range(npt // zrows):
            pltpu.sync_copy(zbuf, aggs.at[pl.ds(sid * npt + k * zrows, zrows)])
        if tail:
            @pl.when(sid == 0)
            def _():
                pltpu.sync_copy(zbuf.at[pl.ds(0, tail)],
                                aggs.at[pl.ds(NS * npt, tail)])
        plsc.subcore_barrier()

        def chunk_body(c, carry):
            base = wid * ew + c * chunk
            pltpu.sync_copy(src_hbm.at[pl.ds(base, chunk)], srcv)
            pltpu.sync_copy(dst_hbm.at[pl.ds(base, chunk)], dstv)
            pltpu.async_copy(h_hbm.at[srcv], rows, sem).wait()
            pltpu.sync_copy(emb_hbm.at[pl.ds(base, chunk), :], embv)

            def edge_body(i, ec):
                for j in range(groups):
                    sl = pl.ds(j * LANES, LANES)
                    rows[i, sl] = jnp.maximum(rows[i, sl] + embv[i, sl], 0.0)
                return ec
            lax.fori_loop(0, chunk, edge_body, 0)
            pltpu.sync_copy(rows, aggs.at[dstv], add=True)
            return carry
        lax.fori_loop(0, nchunk, chunk_body, 0)

        plsc.subcore_barrier()
        for k in range(npt // zrows):
            r0 = sid * npt + k * zrows
            pltpu.sync_copy(aggs.at[pl.ds(r0, zrows)],
                            out_hbm.at[cid, pl.ds(r0, zrows), :])
        if tail:
            @pl.when(sid == 0)
            def _():
                pltpu.sync_copy(aggs.at[pl.ds(NS * npt, tail)],
                                out_hbm.at[cid, pl.ds(NS * npt, tail), :])

    return body(h, emb, src, dst)


def kernel(x, edge_index, edge_attr, W_init, b_init, W_edge, b_edge, eps,
           W1, b1, W2, b2, gamma, beta):
    num_l = W_edge.shape[0]
    src = edge_index[0]
    dst = edge_index[1]

    h = _encode(x, W_init, b_init)
    emb = _edge_emb(edge_attr, W_edge, b_edge, block_e=8000)

    for l in range(num_l):
        agg = _sc_message_pass(h, emb[l], src, dst)
        eps_l = eps[l].reshape(1, 1)
        h = _layer(h, agg, W1[l], b1[l], W2[l], b2[l],
                   gamma[l].reshape(1, -1), beta[l].reshape(1, -1), eps_l)
    return h


# speedup vs baseline: 2.5652x; 1.2318x over previous
import functools

import jax
import jax.numpy as jnp
from jax import lax
from jax.experimental import pallas as pl
from jax.experimental.pallas import tpu as pltpu
from jax.experimental.pallas import tpu_sc as plsc

NC = 2
NS = 16
LANES = 16


def _encode_body(x_ref, w_ref, b_ref, o_ref):
    o_ref[...] = (
        jnp.dot(x_ref[...], w_ref[...], preferred_element_type=jnp.float32)
        + b_ref[...]
    )


def _encode(x, w, b):
    n, d = x.shape
    return pl.pallas_call(
        _encode_body,
        out_shape=jax.ShapeDtypeStruct((n, d), jnp.float32),
    )(x, w, b)


def _edge_emb_body(a_ref, w_ref, b_ref, o_ref):
    o_ref[0] = (
        jnp.dot(a_ref[...], w_ref[0], preferred_element_type=jnp.float32)
        + b_ref[0]
    )


def _edge_emb(edge_attr, w_edge, b_edge, block_e):
    num_l, de, d = w_edge.shape
    e = edge_attr.shape[0]
    grid = (num_l, e // block_e)
    return pl.pallas_call(
        _edge_emb_body,
        grid=grid,
        in_specs=[
            pl.BlockSpec((block_e, de), lambda l, i: (i, 0)),
            pl.BlockSpec((1, de, d), lambda l, i: (l, 0, 0)),
            pl.BlockSpec((1, 1, d), lambda l, i: (l, 0, 0)),
        ],
        out_specs=pl.BlockSpec((1, block_e, d), lambda l, i: (l, i, 0)),
        out_shape=jax.ShapeDtypeStruct((num_l, e, d), jnp.float32),
    )(edge_attr, w_edge, b_edge.reshape(num_l, 1, d))


def _layer_body(h_ref, agg_ref, w1_ref, b1_ref, w2_ref, b2_ref, g_ref,
                bt_ref, eps_ref, o_ref):
    h = h_ref[...]
    agg = agg_ref[0] + agg_ref[1]
    z = h * (1.0 + eps_ref[0, 0]) + agg
    t = jnp.maximum(
        jnp.dot(z, w1_ref[...], preferred_element_type=jnp.float32)
        + b1_ref[...], 0.0)
    t = jnp.dot(t, w2_ref[...], preferred_element_type=jnp.float32) + b2_ref[...]
    mean = jnp.mean(t, axis=0, keepdims=True)
    var = jnp.mean((t - mean) * (t - mean), axis=0, keepdims=True)
    o_ref[...] = (t - mean) * lax.rsqrt(var + 1e-5) * g_ref[...] + bt_ref[...] + h


def _layer(h, agg, w1, b1, w2, b2, gamma, beta, eps_l):
    n, d = h.shape
    return pl.pallas_call(
        _layer_body,
        out_shape=jax.ShapeDtypeStruct((n, d), jnp.float32),
    )(h, agg, w1, b1, w2, b2, gamma, beta, eps_l)


def _sc_message_pass(h, emb, src, dst, chunk=40, nbuf=3):
    n, d = h.shape
    e = src.shape[0]
    nw = NC * NS
    ew = e // nw
    nchunk = ew // chunk
    assert ew % chunk == 0 and chunk % 8 == 0 and chunk <= 128
    npt = (n // NS) // 8 * 8
    tail = n - npt * NS
    zrows = 48
    assert npt % zrows == 0 and tail % 8 == 0 and tail <= zrows
    groups = d // LANES

    mesh = plsc.VectorSubcoreMesh(core_axis_name="c", subcore_axis_name="s")

    @functools.partial(
        pl.kernel,
        out_type=jax.ShapeDtypeStruct((NC, n, d), jnp.float32),
        mesh=mesh,
        scratch_types=[
            [pltpu.VMEM((chunk,), jnp.int32)] * nbuf,
            [pltpu.VMEM((chunk,), jnp.int32)] * nbuf,
            [pltpu.VMEM((chunk, d), jnp.float32)] * nbuf,
            [pltpu.VMEM((chunk, d), jnp.float32)] * nbuf,
            pltpu.VMEM((zrows, d), jnp.float32),
            pltpu.VMEM_SHARED((n, d), jnp.float32),
            [pltpu.SemaphoreType.DMA] * nbuf,
            [pltpu.SemaphoreType.DMA] * nbuf,
            [pltpu.SemaphoreType.DMA] * nbuf,
        ],
    )
    def body(h_hbm, emb_hbm, src_hbm, dst_hbm, out_hbm,
             srcv, dstv, rows, embv, zbuf, aggs, isem, gsem, ssem):
        cid = lax.axis_index("c")
        sid = lax.axis_index("s")
        wid = sid * NC + cid

        def zero_body(i, carry):
            for j in range(groups):
                zbuf[i, pl.ds(j * LANES, LANES)] = jnp.zeros(
                    (LANES,), jnp.float32)
            return carry
        lax.fori_loop(0, zrows, zero_body, 0)
        for k in range(npt // zrows):
            pltpu.sync_copy(zbuf, aggs.at[pl.ds(sid * npt + k * zrows, zrows)])
        if tail:
            @pl.when(sid == 0)
            def _():
                pltpu.sync_copy(zbuf.at[pl.ds(0, tail)],
                                aggs.at[pl.ds(NS * npt, tail)])
        plsc.subcore_barrier()

        def idx_loads(c, b):
            base = wid * ew + c * chunk
            pltpu.async_copy(src_hbm.at[pl.ds(base, chunk)], srcv[b], isem[b])
            pltpu.async_copy(dst_hbm.at[pl.ds(base, chunk)], dstv[b], isem[b])

        def data_loads(c, b):
            base = wid * ew + c * chunk
            pltpu.make_async_copy(src_hbm.at[pl.ds(base, chunk)], srcv[b],
                                  isem[b]).wait()
            pltpu.make_async_copy(dst_hbm.at[pl.ds(base, chunk)], dstv[b],
                                  isem[b]).wait()
            pltpu.async_copy(h_hbm.at[srcv[b]], rows[b], gsem[b])
            pltpu.async_copy(emb_hbm.at[pl.ds(base, chunk), :], embv[b],
                             gsem[b])

        def wait_scatter(b):
            pltpu.make_async_copy(rows[b], aggs.at[dstv[b]], ssem[b]).wait()

        def process(c, b):
            base = wid * ew + c * chunk
            pltpu.make_async_copy(h_hbm.at[srcv[b]], rows[b], gsem[b]).wait()
            pltpu.make_async_copy(emb_hbm.at[pl.ds(base, chunk), :], embv[b],
                                  gsem[b]).wait()

            def edge_body(i, ec):
                for j in range(groups):
                    sl = pl.ds(j * LANES, LANES)
                    rows[b][i, sl] = jnp.maximum(
                        rows[b][i, sl] + embv[b][i, sl], 0.0)
                return ec
            lax.fori_loop(0, chunk, edge_body, 0)
            pltpu.async_copy(rows[b], aggs.at[dstv[b]], ssem[b], add=True)

        idx_loads(0, 0)
        idx_loads(1, 1)
        data_loads(0, 0)

        nmain = (nchunk - 4) // nbuf

        def round_body(g, carry):
            for b in range(nbuf):
                c = g * nbuf + b
                process(c, b)

                @pl.when(c >= 1)
                def _():
                    wait_scatter((b + nbuf - 1) % nbuf)
                idx_loads(c + 2, (b + 2) % nbuf)
                data_loads(c + 1, (b + 1) % nbuf)
            return carry
        lax.fori_loop(0, nmain, round_body, 0)

        for c in range(nmain * nbuf, nchunk):
            process(c, c % nbuf)
            if c >= 1:
                wait_scatter((c - 1) % nbuf)
            if c + 2 < nchunk:
                idx_loads(c + 2, (c + 2) % nbuf)
            if c + 1 < nchunk:
                data_loads(c + 1, (c + 1) % nbuf)
        wait_scatter((nchunk - 1) % nbuf)

        plsc.subcore_barrier()
        for k in range(npt // zrows):
            r0 = sid * npt + k * zrows
            pltpu.sync_copy(aggs.at[pl.ds(r0, zrows)],
                            out_hbm.at[cid, pl.ds(r0, zrows), :])
        if tail:
            @pl.when(sid == 0)
            def _():
                pltpu.sync_copy(aggs.at[pl.ds(NS * npt, tail)],
                                out_hbm.at[cid, pl.ds(NS * npt, tail), :])

    return body(h, emb, src, dst)


def kernel(x, edge_index, edge_attr, W_init, b_init, W_edge, b_edge, eps,
           W1, b1, W2, b2, gamma, beta):
    num_l = W_edge.shape[0]
    src = edge_index[0]
    dst = edge_index[1]

    h = _encode(x, W_init, b_init)
    emb = _edge_emb(edge_attr, W_edge, b_edge, block_e=8000)

    for l in range(num_l):
        agg = _sc_message_pass(h, emb[l], src, dst)
        eps_l = eps[l].reshape(1, 1)
        h = _layer(h, agg, W1[l], b1[l], W2[l], b2[l],
                   gamma[l].reshape(1, -1), beta[l].reshape(1, -1), eps_l)
    return h


# speedup vs baseline: 3.2386x; 1.2625x over previous
import functools

import jax
import jax.numpy as jnp
from jax import lax
from jax.experimental import pallas as pl
from jax.experimental.pallas import tpu as pltpu
from jax.experimental.pallas import tpu_sc as plsc

NC = 2
NS = 16
LANES = 16


def _encode_body(x_ref, w_ref, b_ref, o_ref):
    o_ref[...] = (
        jnp.dot(x_ref[...], w_ref[...], preferred_element_type=jnp.float32)
        + b_ref[...]
    )


def _encode(x, w, b):
    n, d = x.shape
    return pl.pallas_call(
        _encode_body,
        out_shape=jax.ShapeDtypeStruct((n, d), jnp.float32),
    )(x, w, b)


def _edge_emb_body(a_ref, w_ref, b_ref, o_ref):
    o_ref[...] = (
        jnp.dot(a_ref[...], w_ref[...], preferred_element_type=jnp.float32)
        + b_ref[...]
    )


def _edge_emb(edge_attr, w_edge_l, b_edge_l, block_e):
    de, d = w_edge_l.shape
    e = edge_attr.shape[0]
    return pl.pallas_call(
        _edge_emb_body,
        grid=(e // block_e,),
        in_specs=[
            pl.BlockSpec((block_e, de), lambda i: (i, 0)),
            pl.BlockSpec((de, d), lambda i: (0, 0)),
            pl.BlockSpec((1, d), lambda i: (0, 0)),
        ],
        out_specs=pl.BlockSpec((block_e, d), lambda i: (i, 0)),
        out_shape=jax.ShapeDtypeStruct((e, d), jnp.float32),
    )(edge_attr, w_edge_l, b_edge_l.reshape(1, d))


def _layer_body(h_ref, agg_ref, w1_ref, b1_ref, w2_ref, b2_ref, g_ref,
                bt_ref, eps_ref, o_ref):
    h = h_ref[...]
    agg = agg_ref[0] + agg_ref[1]
    z = h * (1.0 + eps_ref[0, 0]) + agg
    t = jnp.maximum(
        jnp.dot(z, w1_ref[...], preferred_element_type=jnp.float32)
        + b1_ref[...], 0.0)
    t = jnp.dot(t, w2_ref[...], preferred_element_type=jnp.float32) + b2_ref[...]
    mean = jnp.mean(t, axis=0, keepdims=True)
    var = jnp.mean((t - mean) * (t - mean), axis=0, keepdims=True)
    o_ref[...] = (t - mean) * lax.rsqrt(var + 1e-5) * g_ref[...] + bt_ref[...] + h


def _layer(h, agg, w1, b1, w2, b2, gamma, beta, eps_l):
    n, d = h.shape
    return pl.pallas_call(
        _layer_body,
        out_shape=jax.ShapeDtypeStruct((n, d), jnp.float32),
    )(h, agg, w1, b1, w2, b2, gamma, beta, eps_l)


def _sc_message_pass(h, emb, src, dst, chunk=40, nbuf=3):
    n, d = h.shape
    e = src.shape[0]
    nw = NC * NS
    ew = e // nw
    nchunk = ew // chunk
    assert ew % chunk == 0 and chunk % 8 == 0 and chunk <= 128
    npt = (n // NS) // 8 * 8
    tail = n - npt * NS
    zrows = 48
    assert npt % zrows == 0 and tail % 8 == 0 and tail <= zrows
    groups = d // LANES

    mesh = plsc.VectorSubcoreMesh(core_axis_name="c", subcore_axis_name="s")

    @functools.partial(
        pl.kernel,
        out_type=jax.ShapeDtypeStruct((NC, n, d), jnp.float32),
        mesh=mesh,
        scratch_types=[
            [pltpu.VMEM((chunk,), jnp.int32)] * nbuf,
            [pltpu.VMEM((chunk,), jnp.int32)] * nbuf,
            [pltpu.VMEM((chunk, d), jnp.float32)] * nbuf,
            [pltpu.VMEM((chunk, d), jnp.float32)] * nbuf,
            pltpu.VMEM((zrows, d), jnp.float32),
            pltpu.VMEM_SHARED((n, d), jnp.float32),
            [pltpu.SemaphoreType.DMA] * nbuf,
            [pltpu.SemaphoreType.DMA] * nbuf,
            [pltpu.SemaphoreType.DMA] * nbuf,
        ],
    )
    def body(h_hbm, emb_hbm, src_hbm, dst_hbm, out_hbm,
             srcv, dstv, rows, embv, zbuf, aggs, isem, gsem, ssem):
        cid = lax.axis_index("c")
        sid = lax.axis_index("s")
        wid = sid * NC + cid

        @plsc.parallel_loop(0, zrows, 1, unroll=4)
        def _(i):
            for j in range(groups):
                zbuf[i, pl.ds(j * LANES, LANES)] = jnp.zeros(
                    (LANES,), jnp.float32)
        for k in range(npt // zrows):
            pltpu.sync_copy(zbuf, aggs.at[pl.ds(sid * npt + k * zrows, zrows)])
        if tail:
            @pl.when(sid == 0)
            def _():
                pltpu.sync_copy(zbuf.at[pl.ds(0, tail)],
                                aggs.at[pl.ds(NS * npt, tail)])
        plsc.subcore_barrier()

        def idx_loads(c, b):
            base = wid * ew + c * chunk
            pltpu.async_copy(src_hbm.at[pl.ds(base, chunk)], srcv[b], isem[b])
            pltpu.async_copy(dst_hbm.at[pl.ds(base, chunk)], dstv[b], isem[b])

        def data_loads(c, b):
            base = wid * ew + c * chunk
            pltpu.make_async_copy(src_hbm.at[pl.ds(base, chunk)], srcv[b],
                                  isem[b]).wait()
            pltpu.make_async_copy(dst_hbm.at[pl.ds(base, chunk)], dstv[b],
                                  isem[b]).wait()
            pltpu.async_copy(h_hbm.at[srcv[b]], rows[b], gsem[b])
            pltpu.async_copy(emb_hbm.at[pl.ds(base, chunk), :], embv[b],
                             gsem[b])

        def wait_scatter(b):
            pltpu.make_async_copy(rows[b], aggs.at[dstv[b]], ssem[b]).wait()

        def process(c, b):
            base = wid * ew + c * chunk
            pltpu.make_async_copy(h_hbm.at[srcv[b]], rows[b], gsem[b]).wait()
            pltpu.make_async_copy(emb_hbm.at[pl.ds(base, chunk), :], embv[b],
                                  gsem[b]).wait()

            @plsc.parallel_loop(0, chunk, 1, unroll=4)
            def _(i):
                for j in range(groups):
                    sl = pl.ds(j * LANES, LANES)
                    rows[b][i, sl] = jnp.maximum(
                        rows[b][i, sl] + embv[b][i, sl], 0.0)
            pltpu.async_copy(rows[b], aggs.at[dstv[b]], ssem[b], add=True)

        idx_loads(0, 0)
        idx_loads(1, 1)
        data_loads(0, 0)

        nmain = (nchunk - 4) // nbuf

        def round_body(g, carry):
            for b in range(nbuf):
                c = g * nbuf + b
                process(c, b)

                @pl.when(c >= 1)
                def _():
                    wait_scatter((b + nbuf - 1) % nbuf)
                idx_loads(c + 2, (b + 2) % nbuf)
                data_loads(c + 1, (b + 1) % nbuf)
            return carry
        lax.fori_loop(0, nmain, round_body, 0)

        for c in range(nmain * nbuf, nchunk):
            process(c, c % nbuf)
            if c >= 1:
                wait_scatter((c - 1) % nbuf)
            if c + 2 < nchunk:
                idx_loads(c + 2, (c + 2) % nbuf)
            if c + 1 < nchunk:
                data_loads(c + 1, (c + 1) % nbuf)
        wait_scatter((nchunk - 1) % nbuf)

        plsc.subcore_barrier()
        for k in range(npt // zrows):
            r0 = sid * npt + k * zrows
            pltpu.sync_copy(aggs.at[pl.ds(r0, zrows)],
                            out_hbm.at[cid, pl.ds(r0, zrows), :])
        if tail:
            @pl.when(sid == 0)
            def _():
                pltpu.sync_copy(aggs.at[pl.ds(NS * npt, tail)],
                                out_hbm.at[cid, pl.ds(NS * npt, tail), :])

    return body(h, emb, src, dst)


def kernel(x, edge_index, edge_attr, W_init, b_init, W_edge, b_edge, eps,
           W1, b1, W2, b2, gamma, beta):
    num_l = W_edge.shape[0]
    src = edge_index[0]
    dst = edge_index[1]

    h = _encode(x, W_init, b_init)

    for l in range(num_l):
        emb_l = _edge_emb(edge_attr, W_edge[l], b_edge[l], block_e=8000)
        agg = _sc_message_pass(h, emb_l, src, dst)
        eps_l = eps[l].reshape(1, 1)
        h = _layer(h, agg, W1[l], b1[l], W2[l], b2[l],
                   gamma[l].reshape(1, -1), beta[l].reshape(1, -1), eps_l)
    return h
